# baseline (speedup 1.0000x reference)
import jax
import jax.numpy as jnp
from jax import lax
from jax.experimental import pallas as pl
from jax.experimental.pallas import tpu as pltpu, tpu_sc as plsc

_RANK = 64
_BATCH = 16384
_NW = 32
_BPW = _BATCH // _NW
_CH = 128
_NCH = _BPW // _CH
_L = 16


def _dpr_body(users_hbm, items_hbm, utab_hbm, itab_hbm, w_hbm, b_hbm,
              mean_hbm, std_hbm, logvar_hbm,
              uidx, iidx, upix, ipix, ubuf0, ubuf1, ibuf0, ibuf1,
              mean_v, std_v, logvar_v, w_v, b_v, sem0, sem1):
    wid = lax.axis_index("s") * 2 + lax.axis_index("c")
    base = wid * _BPW

    pltpu.sync_copy(users_hbm.at[pl.ds(base, _BPW)], uidx)
    pltpu.sync_copy(items_hbm.at[pl.ds(base, _BPW)], iidx)
    pltpu.sync_copy(w_hbm, w_v)
    pltpu.sync_copy(b_hbm, b_v)

    def pidx_step(g, _):
        sl = pl.ds(g * _L, _L)
        upix[sl] = lax.shift_right_logical(uidx[sl], 1)
        ipix[sl] = lax.shift_right_logical(iidx[sl], 1)
        return _

    lax.fori_loop(0, _BPW // _L, pidx_step, 0, unroll=2)

    wm = [w_v[0, pl.ds(k * _L, _L)] for k in range(_RANK // _L)]
    wlv = [w_v[1, pl.ds(k * _L, _L)] for k in range(_RANK // _L)]
    bm = b_v[0, pl.ds(0, _L)]
    blv = b_v[1, pl.ds(0, _L)]
    lane = lax.iota(jnp.int32, _L)
    one = jnp.full((_L,), 1, jnp.int32)
    zero = jnp.zeros((_L,), jnp.float32)
    sems = (sem0, sem1)
    ubufs = (ubuf0, ubuf1)
    ibufs = (ibuf0, ibuf1)

    def issue(ci, buf, sem):
        sl = pl.ds(ci * _CH, _CH)
        pltpu.async_copy(utab_hbm.at[upix.at[sl]], ubufs[buf], sem)
        pltpu.async_copy(itab_hbm.at[ipix.at[sl]], ibufs[buf], sem)

    def drain(buf, sem):
        pltpu.make_async_copy(utab_hbm.at[pl.ds(0, _CH)],
                              ubufs[buf], sem).wait()
        pltpu.make_async_copy(itab_hbm.at[pl.ds(0, _CH)],
                              ibufs[buf], sem).wait()

    def compute(ci, buf):
        ub = ubufs[buf]
        ib = ibufs[buf]
        c0 = ci * _CH

        def blk_step(blk, _):
            r0 = c0 + blk * _L
            su = lax.shift_left(uidx[pl.ds(r0, _L)] & one, 6)
            si = lax.shift_left(iidx[pl.ds(r0, _L)] & one, 6)
            accm = zero
            acclv = zero
            for r in range(_L):
                b = blk * _L + r
                ou = su[r]
                oi = si[r]
                am = None
                alv = None
                for k in range(_RANK // _L):
                    u = ub[b, pl.ds(ou + k * _L, _L)]
                    it = ib[b, pl.ds(oi + k * _L, _L)]
                    inter = u * it
                    tm = inter * wm[k]
                    tlv = inter * wlv[k]
                    am = tm if am is None else am + tm
                    alv = tlv if alv is None else alv + tlv
                sel = lane == r
                accm = jnp.where(sel, jnp.sum(am), accm)
                acclv = jnp.where(sel, jnp.sum(alv), acclv)
            lv = acclv + blv
            mean_v[pl.ds(r0, _L)] = accm + bm
            logvar_v[pl.ds(r0, _L)] = lv
            std_v[pl.ds(r0, _L)] = jnp.exp(0.5 * lv)
            return _

        lax.fori_loop(0, _CH // _L, blk_step, 0)

    issue(0, 0, sem0)
    issue(1, 1, sem1)

    def pipe_step(j, _):
        for p in range(2):
            ci = 2 * j + p
            drain(p, sems[p])
            compute(ci, p)

            @pl.when(ci + 2 < _NCH)
            def _issue_next():
                issue(ci + 2, p, sems[p])
        return _

    lax.fori_loop(0, _NCH // 2, pipe_step, 0)

    pltpu.sync_copy(mean_v, mean_hbm.at[pl.ds(base, _BPW)])
    pltpu.sync_copy(std_v, std_hbm.at[pl.ds(base, _BPW)])
    pltpu.sync_copy(logvar_v, logvar_hbm.at[pl.ds(base, _BPW)])


@jax.jit
def _dpr(users, items, utab2, itab2, w_cat, bv):
    mesh = plsc.VectorSubcoreMesh(core_axis_name="c", subcore_axis_name="s")
    out = jax.ShapeDtypeStruct((_BATCH,), jnp.float32)
    f = pl.kernel(
        _dpr_body,
        out_type=(out, out, out),
        mesh=mesh,
        scratch_types=[
            pltpu.VMEM((_BPW,), jnp.int32),
            pltpu.VMEM((_BPW,), jnp.int32),
            pltpu.VMEM((_BPW,), jnp.int32),
            pltpu.VMEM((_BPW,), jnp.int32),
            pltpu.VMEM((_CH, 2 * _RANK), jnp.float32),
            pltpu.VMEM((_CH, 2 * _RANK), jnp.float32),
            pltpu.VMEM((_CH, 2 * _RANK), jnp.float32),
            pltpu.VMEM((_CH, 2 * _RANK), jnp.float32),
            pltpu.VMEM((_BPW,), jnp.float32),
            pltpu.VMEM((_BPW,), jnp.float32),
            pltpu.VMEM((_BPW,), jnp.float32),
            pltpu.VMEM((2, _RANK), jnp.float32),
            pltpu.VMEM((2, _L), jnp.float32),
            pltpu.SemaphoreType.DMA,
            pltpu.SemaphoreType.DMA,
        ],
        compiler_params=pltpu.CompilerParams(needs_layout_passes=False),
    )
    return f(users, items, utab2, itab2, w_cat, bv)


def _pack_pairs(table):
    return jnp.concatenate([table[0::2], table[1::2]], axis=1)


def kernel(users, items, user_table, item_table, W_mean, b_mean, W_logvar,
           b_logvar):
    utab2 = _pack_pairs(user_table)
    itab2 = _pack_pairs(item_table)
    w_cat = jnp.stack([W_mean.reshape(_RANK), W_logvar.reshape(_RANK)])
    bv = jnp.stack([jnp.full((_L,), b_mean[0], jnp.float32),
                    jnp.full((_L,), b_logvar[0], jnp.float32)])
    mean, std, logvar = _dpr(users, items, utab2, itab2, w_cat, bv)
    return (mean, std, logvar)


# speedup vs baseline: 34.6513x; 34.6513x over previous
import jax
import jax.numpy as jnp
from jax import lax
from jax.experimental import pallas as pl
from jax.experimental.pallas import tpu as pltpu, tpu_sc as plsc

_RANK = 64
_BATCH = 16384
_NW = 32
_BPW = _BATCH // _NW
_CH = 16
_NCH = _BPW // _CH
_L = 16


def _dpr_body(users_hbm, items_hbm, utab_hbm, itab_hbm, w_hbm, b_hbm,
              mean_hbm, std_hbm, logvar_hbm,
              uidx, iidx, ubuf0, ubuf1, ibuf0, ibuf1,
              mean_v, std_v, logvar_v, w_v, b_v, sem0, sem1):
    wid = lax.axis_index("s") * 2 + lax.axis_index("c")
    base = wid * _BPW

    pltpu.sync_copy(users_hbm.at[pl.ds(base, _BPW)], uidx)
    pltpu.sync_copy(items_hbm.at[pl.ds(base, _BPW)], iidx)
    pltpu.sync_copy(w_hbm, w_v)
    pltpu.sync_copy(b_hbm, b_v)

    wm = [w_v[0, pl.ds(k * _L, _L)] for k in range(_RANK // _L)]
    wlv = [w_v[1, pl.ds(k * _L, _L)] for k in range(_RANK // _L)]
    bm = b_v[0, pl.ds(0, _L)]
    blv = b_v[1, pl.ds(0, _L)]
    lane = lax.iota(jnp.int32, _L)
    seven = jnp.full((_L,), 7, jnp.int32)
    zero = jnp.zeros((_L,), jnp.float32)
    sems = (sem0, sem1)
    ubufs = (ubuf0, ubuf1)
    ibufs = (ibuf0, ibuf1)

    def issue(ci, buf, sem):
        ub = ubufs[buf]
        ib = ibufs[buf]
        c0 = ci * _CH
        for g in range(_CH // _L):
            uv = lax.shift_right_logical(uidx[pl.ds(c0 + g * _L, _L)], 3)
            iv = lax.shift_right_logical(iidx[pl.ds(c0 + g * _L, _L)], 3)
            for l in range(_L):
                slot = g * _L + l
                pltpu.async_copy(utab_hbm.at[uv[l]], ub.at[slot], sem)
                pltpu.async_copy(itab_hbm.at[iv[l]], ib.at[slot], sem)

    def drain(buf, sem):
        pltpu.make_async_copy(utab_hbm.at[pl.ds(0, _CH)],
                              ubufs[buf], sem).wait()
        pltpu.make_async_copy(itab_hbm.at[pl.ds(0, _CH)],
                              ibufs[buf], sem).wait()

    def compute(ci, buf):
        ub = ubufs[buf]
        ib = ibufs[buf]
        c0 = ci * _CH
        for g in range(_CH // _L):
            b0 = c0 + g * _L
            su = uidx[pl.ds(b0, _L)] & seven
            si = iidx[pl.ds(b0, _L)] & seven
            accm = zero
            acclv = zero
            for r in range(_L):
                slot = g * _L + r
                am = None
                alv = None
                for k in range(_RANK // _L):
                    u = ub[slot, su[r], pl.ds(k * _L, _L)]
                    it = ib[slot, si[r], pl.ds(k * _L, _L)]
                    inter = u * it
                    tm = inter * wm[k]
                    tlv = inter * wlv[k]
                    am = tm if am is None else am + tm
                    alv = tlv if alv is None else alv + tlv
                sel = lane == r
                accm = jnp.where(sel, jnp.sum(am), accm)
                acclv = jnp.where(sel, jnp.sum(alv), acclv)
            lv = acclv + blv
            mean_v[pl.ds(b0, _L)] = accm + bm
            logvar_v[pl.ds(b0, _L)] = lv
            std_v[pl.ds(b0, _L)] = jnp.exp(0.5 * lv)

    issue(0, 0, sem0)
    issue(1, 1, sem1)

    def pipe_step(j, _):
        for p in range(2):
            ci = 2 * j + p
            drain(p, sems[p])
            compute(ci, p)

            @pl.when(ci + 2 < _NCH)
            def _issue_next():
                issue(ci + 2, p, sems[p])
        return _

    lax.fori_loop(0, _NCH // 2, pipe_step, 0)

    pltpu.sync_copy(mean_v, mean_hbm.at[pl.ds(base, _BPW)])
    pltpu.sync_copy(std_v, std_hbm.at[pl.ds(base, _BPW)])
    pltpu.sync_copy(logvar_v, logvar_hbm.at[pl.ds(base, _BPW)])


@jax.jit
def _dpr(users, items, utab3, itab3, w_cat, bv):
    mesh = plsc.VectorSubcoreMesh(core_axis_name="c", subcore_axis_name="s")
    out = jax.ShapeDtypeStruct((_BATCH,), jnp.float32)
    f = pl.kernel(
        _dpr_body,
        out_type=(out, out, out),
        mesh=mesh,
        scratch_types=[
            pltpu.VMEM((_BPW,), jnp.int32),
            pltpu.VMEM((_BPW,), jnp.int32),
            pltpu.VMEM((_CH, 8, _RANK), jnp.float32),
            pltpu.VMEM((_CH, 8, _RANK), jnp.float32),
            pltpu.VMEM((_CH, 8, _RANK), jnp.float32),
            pltpu.VMEM((_CH, 8, _RANK), jnp.float32),
            pltpu.VMEM((_BPW,), jnp.float32),
            pltpu.VMEM((_BPW,), jnp.float32),
            pltpu.VMEM((_BPW,), jnp.float32),
            pltpu.VMEM((2, _RANK), jnp.float32),
            pltpu.VMEM((2, _L), jnp.float32),
            pltpu.SemaphoreType.DMA,
            pltpu.SemaphoreType.DMA,
        ],
        compiler_params=pltpu.CompilerParams(needs_layout_passes=False),
    )
    return f(users, items, utab3, itab3, w_cat, bv)


def kernel(users, items, user_table, item_table, W_mean, b_mean, W_logvar,
           b_logvar):
    utab3 = user_table.reshape(-1, 8, _RANK)
    itab3 = item_table.reshape(-1, 8, _RANK)
    w_cat = jnp.stack([W_mean.reshape(_RANK), W_logvar.reshape(_RANK)])
    bv = jnp.stack([jnp.full((_L,), b_mean[0], jnp.float32),
                    jnp.full((_L,), b_logvar[0], jnp.float32)])
    mean, std, logvar = _dpr(users, items, utab3, itab3, w_cat, bv)
    return (mean, std, logvar)
